# baseline (speedup 1.0000x reference)
import functools

import jax
import jax.numpy as jnp
from jax import lax
from jax.experimental import pallas as pl
from jax.experimental.pallas import tpu as pltpu
from jax.experimental.pallas import tpu_sc as plsc

D_MODEL = 64
SCALE = 8.0

NW = 32
BLK = 128


def _emb_call(xT, tab2, S, B, D):
    mesh = plsc.VectorSubcoreMesh(core_axis_name="c", subcore_axis_name="s")

    @functools.partial(
        pl.kernel,
        mesh=mesh,
        out_type=jax.ShapeDtypeStruct((S, D, B), jnp.float32),
        scratch_types=[
            pltpu.VMEM((S, BLK), jnp.int32),
            pltpu.VMEM((BLK,), jnp.int32),
            pltpu.VMEM((BLK,), jnp.int32),
            pltpu.VMEM((BLK, 2 * D), jnp.float32),
            pltpu.VMEM((D, BLK), jnp.float32),
            pltpu.SemaphoreType.DMA,
        ],
        compiler_params=pltpu.CompilerParams(needs_layout_passes=False),
    )
    def emb_kernel(xT_hbm, tab_hbm, out_hbm, idx_all, pidx, pcol, rows, oblk, sem):
        w = lax.axis_index("s") * 2 + lax.axis_index("c")
        col0 = w * BLK
        pltpu.sync_copy(xT_hbm.at[:, pl.ds(col0, BLK)], idx_all)

        def s_body(s, carry):
            for g in range(BLK // 16):
                sl = pl.ds(g * 16, 16)
                v = idx_all[s, sl]
                pidx[sl] = lax.shift_right_logical(v, 1)
                pcol[sl] = (v & 1) * D
            pltpu.async_copy(tab_hbm.at[pidx], rows, sem).wait()

            def f_body(f, c2):
                for g in range(BLK // 16):
                    sl = pl.ds(g * 16, 16)
                    bvec = lax.iota(jnp.int32, 16) + (g * 16)
                    col = pcol[sl] + f
                    oblk[f, sl] = plsc.load_gather(rows, [bvec, col]) * SCALE
                return c2

            lax.fori_loop(0, D, f_body, 0)
            pltpu.sync_copy(oblk, out_hbm.at[s, :, pl.ds(col0, BLK)])
            return carry

        lax.fori_loop(0, S, s_body, 0)

    return emb_kernel(xT, tab2)


def kernel(x, table):
    B, S = x.shape
    V, D = table.shape
    xT = x.astype(jnp.int32).T
    tab2 = table.reshape(V // 2, 2 * D)
    out_p = _emb_call(xT, tab2, S, B, D)
    return out_p.transpose(2, 0, 1)


# speedup vs baseline: 1.5331x; 1.5331x over previous
import functools

import jax
import jax.numpy as jnp
from jax import lax
from jax.experimental import pallas as pl
from jax.experimental.pallas import tpu as pltpu
from jax.experimental.pallas import tpu_sc as plsc

D_MODEL = 64
SCALE = 8.0

NW = 32
BLK = 128
NG = BLK // 16


def _emb_call(xT, tab2, S, B, D):
    mesh = plsc.VectorSubcoreMesh(core_axis_name="c", subcore_axis_name="s")

    @functools.partial(
        pl.kernel,
        mesh=mesh,
        out_type=jax.ShapeDtypeStruct((S, D, B), jnp.float32),
        scratch_types=[
            pltpu.VMEM((S, BLK), jnp.int32),
            pltpu.VMEM((BLK,), jnp.int32),
            pltpu.VMEM((BLK,), jnp.int32),
            pltpu.VMEM((BLK, 2 * D), jnp.float32),
            pltpu.VMEM((BLK, 2 * D), jnp.float32),
            pltpu.VMEM((D, BLK), jnp.float32),
            pltpu.VMEM((D, BLK), jnp.float32),
            pltpu.SemaphoreType.DMA,
            pltpu.SemaphoreType.DMA,
            pltpu.SemaphoreType.DMA,
            pltpu.SemaphoreType.DMA,
        ],
        compiler_params=pltpu.CompilerParams(needs_layout_passes=False),
    )
    def emb_kernel(xT_hbm, tab_hbm, out_hbm, idx_all,
                   pidx0, pidx1, rows0, rows1, oblk0, oblk1,
                   gsem0, gsem1, osem0, osem1):
        w = lax.axis_index("s") * 2 + lax.axis_index("c")
        col0 = w * BLK
        pltpu.sync_copy(xT_hbm.at[:, pl.ds(col0, BLK)], idx_all)

        bvec = [lax.iota(jnp.int32, 16) + (g * 16) for g in range(NG)]

        def prep_fire(s, pidx, rows, gsem):
            for g in range(NG):
                sl = pl.ds(g * 16, 16)
                pidx[sl] = lax.shift_right_logical(idx_all[s, sl], 1)
            pltpu.async_copy(tab_hbm.at[pidx], rows, gsem)

        def wait_gather(pidx, rows, gsem):
            pltpu.make_async_copy(tab_hbm.at[pidx], rows, gsem).wait()

        def extract(s, rows, oblk):
            pcol = [(idx_all[s, pl.ds(g * 16, 16)] & 1) << 6 for g in range(NG)]

            def f_body(f, c2):
                for g in range(NG):
                    vals = plsc.load_gather(rows, [bvec[g], pcol[g] + f])
                    oblk[f, pl.ds(g * 16, 16)] = vals * SCALE
                return c2

            lax.fori_loop(0, D, f_body, 0, unroll=8)

        def out_dma(s, oblk, osem):
            return pltpu.make_async_copy(
                oblk, out_hbm.at[s, :, pl.ds(col0, BLK)], osem)

        prep_fire(0, pidx0, rows0, gsem0)

        def pair_body(i, carry):
            sA = 2 * i
            sB = sA + 1
            prep_fire(sB, pidx1, rows1, gsem1)
            wait_gather(pidx0, rows0, gsem0)

            @pl.when(i > 0)
            def _():
                out_dma(sA, oblk0, osem0).wait()

            extract(sA, rows0, oblk0)
            out_dma(sA, oblk0, osem0).start()

            @pl.when(i < S // 2 - 1)
            def _():
                prep_fire(sA + 2, pidx0, rows0, gsem0)

            wait_gather(pidx1, rows1, gsem1)

            @pl.when(i > 0)
            def _():
                out_dma(sB, oblk1, osem1).wait()

            extract(sB, rows1, oblk1)
            out_dma(sB, oblk1, osem1).start()
            return carry

        lax.fori_loop(0, S // 2, pair_body, 0)
        out_dma(S - 2, oblk0, osem0).wait()
        out_dma(S - 1, oblk1, osem1).wait()

    return emb_kernel(xT, tab2)


def kernel(x, table):
    B, S = x.shape
    V, D = table.shape
    xT = x.astype(jnp.int32).T
    tab2 = table.reshape(V // 2, 2 * D)
    out_p = _emb_call(xT, tab2, S, B, D)
    return out_p.transpose(2, 0, 1)


# speedup vs baseline: 2.3797x; 1.5522x over previous
import functools

import jax
import jax.numpy as jnp
from jax import lax
from jax.experimental import pallas as pl
from jax.experimental.pallas import tpu as pltpu
from jax.experimental.pallas import tpu_sc as plsc

D_MODEL = 64
SCALE = 8.0

NW = 32
BLK = 128
NG = BLK // 16


def _emb_call(xT, tab2, S, B, D):
    mesh = plsc.VectorSubcoreMesh(core_axis_name="c", subcore_axis_name="s")

    @functools.partial(
        pl.kernel,
        mesh=mesh,
        out_type=jax.ShapeDtypeStruct((S, D, B), jnp.float32),
        scratch_types=[
            pltpu.VMEM((S, BLK), jnp.int32),
            pltpu.VMEM((BLK,), jnp.int32),
            pltpu.VMEM((BLK,), jnp.int32),
            pltpu.VMEM((BLK, 2 * D), jnp.float32),
            pltpu.VMEM((BLK, 2 * D), jnp.float32),
            pltpu.VMEM((D, BLK), jnp.float32),
            pltpu.VMEM((D, BLK), jnp.float32),
            pltpu.SemaphoreType.DMA,
            pltpu.SemaphoreType.DMA,
            pltpu.SemaphoreType.DMA,
            pltpu.SemaphoreType.DMA,
        ],
        compiler_params=pltpu.CompilerParams(needs_layout_passes=False),
    )
    def emb_kernel(xT_hbm, tab_hbm, out_hbm, idx_all,
                   pidx0, pidx1, rows0, rows1, oblk0, oblk1,
                   gsem0, gsem1, osem0, osem1):
        w = lax.axis_index("s") * 2 + lax.axis_index("c")
        col0 = w * BLK
        pltpu.sync_copy(xT_hbm.at[:, pl.ds(col0, BLK)], idx_all)

        bvec = [lax.iota(jnp.int32, 16) + (g * 16) for g in range(NG)]

        def prep_fire(s, pidx, rows, gsem):
            for g in range(NG):
                sl = pl.ds(g * 16, 16)
                pidx[sl] = lax.shift_right_logical(idx_all[s, sl], 1)
            pltpu.async_copy(tab_hbm.at[pidx], rows, gsem)

        def wait_gather(pidx, rows, gsem):
            pltpu.make_async_copy(tab_hbm.at[pidx], rows, gsem).wait()

        def extract(s, rows, oblk):
            pcol = tuple(
                (idx_all[s, pl.ds(g * 16, 16)] & 1) << 6 for g in range(NG))

            @plsc.parallel_loop(0, D, unroll=8)
            def _(f):
                for g in range(NG):
                    vals = plsc.load_gather(rows, [bvec[g], pcol[g] + f])
                    oblk[f, pl.ds(g * 16, 16)] = vals * SCALE

        def out_dma(s, oblk, osem):
            return pltpu.make_async_copy(
                oblk, out_hbm.at[s, :, pl.ds(col0, BLK)], osem)

        prep_fire(0, pidx0, rows0, gsem0)

        def pair_body(i, carry):
            sA = 2 * i
            sB = sA + 1
            prep_fire(sB, pidx1, rows1, gsem1)
            wait_gather(pidx0, rows0, gsem0)

            @pl.when(i > 0)
            def _():
                out_dma(sA, oblk0, osem0).wait()

            extract(sA, rows0, oblk0)
            out_dma(sA, oblk0, osem0).start()

            @pl.when(i < S // 2 - 1)
            def _():
                prep_fire(sA + 2, pidx0, rows0, gsem0)

            wait_gather(pidx1, rows1, gsem1)

            @pl.when(i > 0)
            def _():
                out_dma(sB, oblk1, osem1).wait()

            extract(sB, rows1, oblk1)
            out_dma(sB, oblk1, osem1).start()
            return carry

        lax.fori_loop(0, S // 2, pair_body, 0)
        out_dma(S - 2, oblk0, osem0).wait()
        out_dma(S - 1, oblk1, osem1).wait()

    return emb_kernel(xT, tab2)


def kernel(x, table):
    B, S = x.shape
    V, D = table.shape
    xT = x.astype(jnp.int32).T
    tab2 = table.reshape(V // 2, 2 * D)
    out_p = _emb_call(xT, tab2, S, B, D)
    return out_p.transpose(2, 0, 1)
